# baseline (speedup 1.0000x reference)
import functools

import jax
import jax.numpy as jnp
from jax import lax
from jax.experimental import pallas as pl
from jax.experimental.pallas import tpu as pltpu
from jax.experimental.pallas import tpu_sc as plsc

Q, D, K = 4096, 16, 1024
L = 16
NC, NS = 2, 16
NW = NC * NS
QPW = Q // NW
CHUNKS = K // L
QBLK = 2
CUNROLL = 8

_mesh = plsc.VectorSubcoreMesh(core_axis_name="c", subcore_axis_name="s")


@functools.partial(
    pl.kernel,
    out_type=[
        jax.ShapeDtypeStruct((Q,), jnp.int32),
        jax.ShapeDtypeStruct((NW, L), jnp.float32),
    ],
    mesh=_mesh,
    compiler_params=pltpu.CompilerParams(needs_layout_passes=False),
    scratch_types=[
        pltpu.VMEM((QPW, D), jnp.float32),
        pltpu.VMEM((K * D,), jnp.float32),
        pltpu.VMEM((D, K), jnp.float32),
        pltpu.VMEM((QPW,), jnp.int32),
        pltpu.VMEM((L,), jnp.float32),
    ],
)
def _sc_kernel(e_hbm, c_hbm, assign_hbm, part_hbm,
               e_v, c_v, ct_v, idx_v, pv_v):
    cid = lax.axis_index("c")
    sid = lax.axis_index("s")
    wid = sid * NC + cid
    base = wid * QPW

    pltpu.sync_copy(e_hbm.at[pl.ds(base, QPW), :], e_v)
    pltpu.sync_copy(c_hbm, c_v)

    iota = lax.iota(jnp.int32, L)
    gdn = lax.GatherDimensionNumbers(
        offset_dims=(), collapsed_slice_dims=(0,), start_index_map=(0,))

    def lane_splat(vec, d):
        idx = jnp.full((L, 1), d, jnp.int32)
        return lax.gather(vec, idx, gdn, slice_sizes=(1,),
                          mode=lax.GatherScatterMode.PROMISE_IN_BOUNDS)

    iota_d = iota * D
    for c in range(CHUNKS):
        rows_d = iota_d + c * L * D
        for d in range(D):
            col = plsc.load_gather(c_v, [rows_d + d])
            ct_v[d, pl.ds(c * L, L)] = col

    neg = jnp.full((L,), -1.0, jnp.float32)
    zero_i = jnp.zeros((L,), jnp.int32)
    big_i = jnp.full((L,), K, jnp.int32)
    zero_f = jnp.zeros((L,), jnp.float32)
    GPS = L // QBLK

    def q_group(g, outer_carry):
        lacc, idxvec = outer_carry
        qs = [g * QBLK + j for j in range(QBLK)]
        splats = []
        for q in qs:
            ev = e_v[q, :]
            splats.append([lane_splat(ev, d) for d in range(D)])

        def chunk_oct(c8, carry):
            mvs = list(carry[0])
            mis = list(carry[1])
            for cc in range(CUNROLL):
                cbase = (c8 * CUNROLL + cc) * L
                idxc = iota + cbase
                cts = [ct_v[d, pl.ds(cbase, L)] for d in range(D)]
                for j in range(QBLK):
                    a = None
                    for d in range(D):
                        diff = cts[d] - splats[j][d]
                        sq = diff * diff
                        a = sq if a is None else a + sq
                    m = a > mvs[j]
                    mvs[j] = jnp.where(m, a, mvs[j])
                    mis[j] = jnp.where(m, idxc, mis[j])
            return (tuple(mvs), tuple(mis))

        carry0 = (tuple(neg for _ in range(QBLK)),
                  tuple(zero_i for _ in range(QBLK)))
        mvs, mis = lax.fori_loop(0, CHUNKS // CUNROLL, chunk_oct, carry0)

        for j in range(QBLK):
            jj = (g % GPS) * QBLK + j
            maxd = jnp.max(mvs[j])
            lacc = lacc + jnp.where(iota == jj, maxd, zero_f)
            cand = jnp.where(mvs[j] == maxd, mis[j], big_i)
            idxvec = jnp.where(iota == jj, jnp.min(cand), idxvec)

        @pl.when(g % GPS == GPS - 1)
        def _store():
            idx_v[pl.ds((g // GPS) * L, L)] = idxvec

        return (lacc, idxvec)

    lacc, _ = lax.fori_loop(0, QPW // QBLK, q_group, (zero_f, zero_i))

    total = jnp.sum(lacc)
    pv_v[...] = jnp.where(iota == 0, total, zero_f)

    pltpu.sync_copy(idx_v, assign_hbm.at[pl.ds(base, QPW)])
    pltpu.sync_copy(pv_v, part_hbm.at[wid])


def kernel(embeddings, centroids):
    assignments, partials = _sc_kernel(embeddings, centroids.reshape(-1))
    loss = jnp.sum(partials)
    return (loss, assignments)
